# baseline (device time: 2412596 ns/iter reference)
import jax
import jax.numpy as jnp
from jax import lax
from jax.experimental import pallas as pl
from jax.experimental.pallas import tpu as pltpu

M = 32768
N = 1024
HALF = M // 2
CHMAX = 256
S = 6
LOOK = 3

SIZES = [256] * 64
assert sum(SIZES) == HALF
OFFS = [sum(SIZES[:i]) for i in range(len(SIZES))]
K = len(SIZES)


def _kernel_main(x):
    def body(x_hbm, out_hbm, f32buf, my16, p1rcv, acc,
             ld_sems, st_sems, p1s, p1r, p2s, p2r):
        my_x = lax.axis_index("x")
        my_y = lax.axis_index("y")
        peer_y = (my_x, 1 - my_y)
        peer_x = (1 - my_x, my_y)

        def mine(j):
            return pl.ds(my_x * HALF + OFFS[j], SIZES[j])

        def theirs(j):
            return pl.ds((1 - my_x) * HALF + OFFS[j], SIZES[j])

        def load(j):
            return pltpu.make_async_copy(
                x_hbm.at[mine(j)],
                f32buf.at[j % 2, pl.ds(0, SIZES[j])],
                ld_sems.at[j % 2])

        def store(j):
            return pltpu.make_async_copy(
                acc.at[j % S, pl.ds(0, SIZES[j])],
                out_hbm.at[mine(j)],
                st_sems.at[j % S])

        def p1(j):
            return pltpu.make_async_remote_copy(
                src_ref=my16.at[j % S, pl.ds(0, SIZES[j])],
                dst_ref=p1rcv.at[j % S, pl.ds(0, SIZES[j])],
                send_sem=p1s.at[j % S], recv_sem=p1r.at[j % S],
                device_id=peer_y, device_id_type=pl.DeviceIdType.MESH)

        def p2(j):
            return pltpu.make_async_remote_copy(
                src_ref=acc.at[j % S, pl.ds(0, SIZES[j])],
                dst_ref=out_hbm.at[mine(j)],
                send_sem=p2s.at[j % S], recv_sem=p2r.at[j % S],
                device_id=peer_x, device_id_type=pl.DeviceIdType.MESH)

        def p2_recv(j):
            return pltpu.make_async_remote_copy(
                src_ref=acc.at[j % S, pl.ds(0, SIZES[j])],
                dst_ref=out_hbm.at[theirs(j)],
                send_sem=p2s.at[j % S], recv_sem=p2r.at[j % S],
                device_id=peer_x, device_id_type=pl.DeviceIdType.MESH)

        def cast(j):
            sz = SIZES[j]
            my16[j % S, :sz] = f32buf[j % 2, :sz].astype(jnp.bfloat16)

        barrier_sem = pltpu.get_barrier_semaphore()
        for nbr in (peer_y, peer_x):
            pl.semaphore_signal(barrier_sem, inc=1, device_id=nbr,
                                device_id_type=pl.DeviceIdType.MESH)
        pl.semaphore_wait(barrier_sem, 2)

        load(0).start()
        load(1).start()
        for j in range(LOOK):
            load(j).wait()
            cast(j)
            p1(j).start()
            if j + 2 < K:
                load(j + 2).start()

        for k in range(K):
            s = k % S
            sz = SIZES[k]
            p1(k).wait_recv()
            if k >= S:
                p2(k - S).wait_send()
                store(k - S).wait()
            acc[s, :sz] = my16[s, :sz] + p1rcv[s, :sz]
            p2(k).start()
            store(k).start()
            if k + LOOK < K:
                load(k + LOOK).wait()
                if k + LOOK - S >= 0:
                    p1(k + LOOK - S).wait_send()
                cast(k + LOOK)
                p1(k + LOOK).start()
                if k + LOOK + 2 < K:
                    load(k + LOOK + 2).start()
            if k >= 1:
                p2_recv(k - 1).wait_recv()

        p2_recv(K - 1).wait_recv()
        for j in range(K - S, K):
            p2(j).wait_send()
            store(j).wait()
            p1(j).wait_send()

    return pl.pallas_call(
        body,
        out_shape=jax.ShapeDtypeStruct((M, N), jnp.bfloat16),
        in_specs=[pl.BlockSpec(memory_space=pl.ANY)],
        out_specs=pl.BlockSpec(memory_space=pltpu.MemorySpace.HBM),
        scratch_shapes=[
            pltpu.VMEM((2, CHMAX, N), jnp.float32),
            pltpu.VMEM((S, CHMAX, N), jnp.bfloat16),
            pltpu.VMEM((S, CHMAX, N), jnp.bfloat16),
            pltpu.VMEM((S, CHMAX, N), jnp.bfloat16),
            pltpu.SemaphoreType.DMA((2,)),
            pltpu.SemaphoreType.DMA((S,)),
            pltpu.SemaphoreType.DMA((S,)),
            pltpu.SemaphoreType.DMA((S,)),
            pltpu.SemaphoreType.DMA((S,)),
            pltpu.SemaphoreType.DMA((S,)),
        ],
        compiler_params=pltpu.CompilerParams(
            collective_id=0, vmem_limit_bytes=64 * 1024 * 1024),
    )(x)


def _copy_out(y):
    NCH = 16
    ROWS = M // NCH

    def body(y_hbm, o_hbm, sems):
        cps = []
        for k in range(NCH):
            rows = pl.ds(k * ROWS, ROWS)
            c = pltpu.make_async_copy(y_hbm.at[rows], o_hbm.at[rows],
                                      sems.at[k])
            c.start()
            cps.append(c)
        for c in cps:
            c.wait()

    return pl.pallas_call(
        body,
        out_shape=jax.ShapeDtypeStruct((M, N), jnp.bfloat16),
        in_specs=[pl.BlockSpec(memory_space=pl.ANY)],
        out_specs=pl.BlockSpec(memory_space=pl.ANY),
        scratch_shapes=[pltpu.SemaphoreType.DMA((NCH,))],
    )(y)


def kernel(x):
    return _copy_out(_kernel_main(x))


# device time: 415194 ns/iter; 5.8108x vs baseline; 5.8108x over previous
import jax
import jax.numpy as jnp
from jax import lax
from jax.experimental import pallas as pl
from jax.experimental.pallas import tpu as pltpu

M = 32768
N = 1024
HALF = M // 2
CHMAX = 256
S = 6
LOOK = 3

SIZES = [256] * 64
assert sum(SIZES) == HALF
OFFS = [sum(SIZES[:i]) for i in range(len(SIZES))]
K = len(SIZES)


def _kernel_main(x):
    def body(x_hbm, out_hbm, f32buf, my16, p1rcv, acc,
             ld_sems, st_sems, p1s, p1r, p2s, p2r):
        my_x = lax.axis_index("x")
        my_y = lax.axis_index("y")
        peer_y = (my_x, 1 - my_y)
        peer_x = (1 - my_x, my_y)

        def mine(j):
            return pl.ds(my_x * HALF + OFFS[j], SIZES[j])

        def theirs(j):
            return pl.ds((1 - my_x) * HALF + OFFS[j], SIZES[j])

        def load(j):
            return pltpu.make_async_copy(
                x_hbm.at[mine(j)],
                f32buf.at[j % 2, pl.ds(0, SIZES[j])],
                ld_sems.at[j % 2])

        def store(j):
            return pltpu.make_async_copy(
                acc.at[j % S, pl.ds(0, SIZES[j])],
                out_hbm.at[mine(j)],
                st_sems.at[j % S])

        def p1(j):
            return pltpu.make_async_remote_copy(
                src_ref=my16.at[j % S, pl.ds(0, SIZES[j])],
                dst_ref=p1rcv.at[j % S, pl.ds(0, SIZES[j])],
                send_sem=p1s.at[j % S], recv_sem=p1r.at[j % S],
                device_id=peer_y, device_id_type=pl.DeviceIdType.MESH)

        def p2(j):
            return pltpu.make_async_remote_copy(
                src_ref=acc.at[j % S, pl.ds(0, SIZES[j])],
                dst_ref=out_hbm.at[mine(j)],
                send_sem=p2s.at[j % S], recv_sem=p2r.at[j % S],
                device_id=peer_x, device_id_type=pl.DeviceIdType.MESH)

        def p2_recv(j):
            return pltpu.make_async_remote_copy(
                src_ref=acc.at[j % S, pl.ds(0, SIZES[j])],
                dst_ref=out_hbm.at[theirs(j)],
                send_sem=p2s.at[j % S], recv_sem=p2r.at[j % S],
                device_id=peer_x, device_id_type=pl.DeviceIdType.MESH)

        def cast(j):
            sz = SIZES[j]
            my16[j % S, :sz] = f32buf[j % 2, :sz].astype(jnp.bfloat16)

        barrier_sem = pltpu.get_barrier_semaphore()
        for nbr in (peer_y, peer_x):
            pl.semaphore_signal(barrier_sem, inc=1, device_id=nbr,
                                device_id_type=pl.DeviceIdType.MESH)
        pl.semaphore_wait(barrier_sem, 2)

        load(0).start()
        load(1).start()
        for j in range(LOOK):
            load(j).wait()
            cast(j)
            p1(j).start()
            if j + 2 < K:
                load(j + 2).start()

        for k in range(K):
            s = k % S
            sz = SIZES[k]
            p1(k).wait_recv()
            if k >= S:
                p2(k - S).wait_send()
                store(k - S).wait()
            acc[s, :sz] = my16[s, :sz] + p1rcv[s, :sz]
            p2(k).start()
            store(k).start()
            if k + LOOK < K:
                load(k + LOOK).wait()
                if k + LOOK - S >= 0:
                    p1(k + LOOK - S).wait_send()
                cast(k + LOOK)
                p1(k + LOOK).start()
                if k + LOOK + 2 < K:
                    load(k + LOOK + 2).start()
            if k >= 1:
                p2_recv(k - 1).wait_recv()

        p2_recv(K - 1).wait_recv()
        for j in range(K - S, K):
            p2(j).wait_send()
            store(j).wait()
            p1(j).wait_send()

    return pl.pallas_call(
        body,
        out_shape=jax.ShapeDtypeStruct((M, N), jnp.bfloat16),
        in_specs=[pl.BlockSpec(memory_space=pl.ANY)],
        out_specs=pl.BlockSpec(memory_space=pltpu.MemorySpace.HBM),
        scratch_shapes=[
            pltpu.VMEM((2, CHMAX, N), jnp.float32),
            pltpu.VMEM((S, CHMAX, N), jnp.bfloat16),
            pltpu.VMEM((S, CHMAX, N), jnp.bfloat16),
            pltpu.VMEM((S, CHMAX, N), jnp.bfloat16),
            pltpu.SemaphoreType.DMA((2,)),
            pltpu.SemaphoreType.DMA((S,)),
            pltpu.SemaphoreType.DMA((S,)),
            pltpu.SemaphoreType.DMA((S,)),
            pltpu.SemaphoreType.DMA((S,)),
            pltpu.SemaphoreType.DMA((S,)),
        ],
        compiler_params=pltpu.CompilerParams(
            collective_id=0, vmem_limit_bytes=64 * 1024 * 1024),
    )(x)


def _copy_out(y):
    NCH = 32
    ROWS = M // NCH
    SC = 8

    def body(y_hbm, o_hbm, buf, ld_sems, st_sems):
        def ld(j):
            return pltpu.make_async_copy(
                y_hbm.at[pl.ds(j * ROWS, ROWS)], buf.at[j % SC],
                ld_sems.at[j % SC])

        def st(j):
            return pltpu.make_async_copy(
                buf.at[j % SC], o_hbm.at[pl.ds(j * ROWS, ROWS)],
                st_sems.at[j % SC])

        for j in range(SC):
            ld(j).start()
        for k in range(NCH):
            ld(k).wait()
            st(k).start()
            m = k + 4
            if SC <= m < NCH:
                st(m - SC).wait()
                ld(m).start()
        for j in range(NCH - SC, NCH):
            st(j).wait()

    return pl.pallas_call(
        body,
        out_shape=jax.ShapeDtypeStruct((M, N), jnp.bfloat16),
        in_specs=[pl.BlockSpec(memory_space=pl.ANY)],
        out_specs=pl.BlockSpec(memory_space=pl.ANY),
        scratch_shapes=[
            pltpu.VMEM((SC, ROWS, N), jnp.bfloat16),
            pltpu.SemaphoreType.DMA((SC,)),
            pltpu.SemaphoreType.DMA((SC,)),
        ],
        compiler_params=pltpu.CompilerParams(
            vmem_limit_bytes=64 * 1024 * 1024),
    )(y)


def kernel(x):
    return _copy_out(_kernel_main(x))


# device time: 414928 ns/iter; 5.8145x vs baseline; 1.0006x over previous
import jax
import jax.numpy as jnp
from jax import lax
from jax.experimental import pallas as pl
from jax.experimental.pallas import tpu as pltpu

M = 32768
N = 1024
HALF = M // 2
CHMAX = 256
S = 8
LOOK = 3

SIZES = [128, 128] + [256] * 62 + [128, 128]
assert sum(SIZES) == HALF
OFFS = [sum(SIZES[:i]) for i in range(len(SIZES))]
K = len(SIZES)


def _kernel_main(x):
    def body(x_hbm, out_hbm, f32buf, my16, p1rcv, acc,
             ld_sems, st_sems, p1s, p1r, p2s, p2r):
        my_x = lax.axis_index("x")
        my_y = lax.axis_index("y")
        peer_y = (my_x, 1 - my_y)
        peer_x = (1 - my_x, my_y)

        def mine(j):
            return pl.ds(my_x * HALF + OFFS[j], SIZES[j])

        def theirs(j):
            return pl.ds((1 - my_x) * HALF + OFFS[j], SIZES[j])

        def load(j):
            return pltpu.make_async_copy(
                x_hbm.at[mine(j)],
                f32buf.at[j % 2, pl.ds(0, SIZES[j])],
                ld_sems.at[j % 2])

        def store(j):
            return pltpu.make_async_copy(
                acc.at[j % S, pl.ds(0, SIZES[j])],
                out_hbm.at[mine(j)],
                st_sems.at[j % S])

        def p1(j):
            return pltpu.make_async_remote_copy(
                src_ref=my16.at[j % S, pl.ds(0, SIZES[j])],
                dst_ref=p1rcv.at[j % S, pl.ds(0, SIZES[j])],
                send_sem=p1s.at[j % S], recv_sem=p1r.at[j % S],
                device_id=peer_y, device_id_type=pl.DeviceIdType.MESH)

        def p2(j):
            return pltpu.make_async_remote_copy(
                src_ref=acc.at[j % S, pl.ds(0, SIZES[j])],
                dst_ref=out_hbm.at[mine(j)],
                send_sem=p2s.at[j % S], recv_sem=p2r.at[j % S],
                device_id=peer_x, device_id_type=pl.DeviceIdType.MESH)

        def p2_recv(j):
            return pltpu.make_async_remote_copy(
                src_ref=acc.at[j % S, pl.ds(0, SIZES[j])],
                dst_ref=out_hbm.at[theirs(j)],
                send_sem=p2s.at[j % S], recv_sem=p2r.at[j % S],
                device_id=peer_x, device_id_type=pl.DeviceIdType.MESH)

        def cast(j):
            sz = SIZES[j]
            my16[j % S, :sz] = f32buf[j % 2, :sz].astype(jnp.bfloat16)

        barrier_sem = pltpu.get_barrier_semaphore()
        for nbr in (peer_y, peer_x):
            pl.semaphore_signal(barrier_sem, inc=1, device_id=nbr,
                                device_id_type=pl.DeviceIdType.MESH)
        pl.semaphore_wait(barrier_sem, 2)

        load(0).start()
        load(1).start()
        for j in range(LOOK):
            load(j).wait()
            cast(j)
            p1(j).start()
            if j + 2 < K:
                load(j + 2).start()

        for k in range(K):
            s = k % S
            sz = SIZES[k]
            p1(k).wait_recv()
            if k >= S:
                p2(k - S).wait_send()
                store(k - S).wait()
            acc[s, :sz] = my16[s, :sz] + p1rcv[s, :sz]
            p2(k).start()
            store(k).start()
            if k + LOOK < K:
                load(k + LOOK).wait()
                if k + LOOK - S >= 0:
                    p1(k + LOOK - S).wait_send()
                cast(k + LOOK)
                p1(k + LOOK).start()
                if k + LOOK + 2 < K:
                    load(k + LOOK + 2).start()
            if k >= 1:
                p2_recv(k - 1).wait_recv()

        p2_recv(K - 1).wait_recv()
        for j in range(K - S, K):
            p2(j).wait_send()
            store(j).wait()
            p1(j).wait_send()

    return pl.pallas_call(
        body,
        out_shape=jax.ShapeDtypeStruct((M, N), jnp.bfloat16),
        in_specs=[pl.BlockSpec(memory_space=pl.ANY)],
        out_specs=pl.BlockSpec(memory_space=pltpu.MemorySpace.HBM),
        scratch_shapes=[
            pltpu.VMEM((2, CHMAX, N), jnp.float32),
            pltpu.VMEM((S, CHMAX, N), jnp.bfloat16),
            pltpu.VMEM((S, CHMAX, N), jnp.bfloat16),
            pltpu.VMEM((S, CHMAX, N), jnp.bfloat16),
            pltpu.SemaphoreType.DMA((2,)),
            pltpu.SemaphoreType.DMA((S,)),
            pltpu.SemaphoreType.DMA((S,)),
            pltpu.SemaphoreType.DMA((S,)),
            pltpu.SemaphoreType.DMA((S,)),
            pltpu.SemaphoreType.DMA((S,)),
        ],
        compiler_params=pltpu.CompilerParams(
            collective_id=0, vmem_limit_bytes=64 * 1024 * 1024),
    )(x)


def _copy_out(y):
    NCH = 32
    ROWS = M // NCH
    SC = 8

    def body(y_hbm, o_hbm, buf, ld_sems, st_sems):
        def ld(j):
            return pltpu.make_async_copy(
                y_hbm.at[pl.ds(j * ROWS, ROWS)], buf.at[j % SC],
                ld_sems.at[j % SC])

        def st(j):
            return pltpu.make_async_copy(
                buf.at[j % SC], o_hbm.at[pl.ds(j * ROWS, ROWS)],
                st_sems.at[j % SC])

        for j in range(SC):
            ld(j).start()
        for k in range(NCH):
            ld(k).wait()
            st(k).start()
            m = k + 4
            if SC <= m < NCH:
                st(m - SC).wait()
                ld(m).start()
        for j in range(NCH - SC, NCH):
            st(j).wait()

    return pl.pallas_call(
        body,
        out_shape=jax.ShapeDtypeStruct((M, N), jnp.bfloat16),
        in_specs=[pl.BlockSpec(memory_space=pl.ANY)],
        out_specs=pl.BlockSpec(memory_space=pl.ANY),
        scratch_shapes=[
            pltpu.VMEM((SC, ROWS, N), jnp.bfloat16),
            pltpu.SemaphoreType.DMA((SC,)),
            pltpu.SemaphoreType.DMA((SC,)),
        ],
        compiler_params=pltpu.CompilerParams(
            vmem_limit_bytes=64 * 1024 * 1024),
    )(y)


def kernel(x):
    return _copy_out(_kernel_main(x))
